# baseline (speedup 1.0000x reference)
import functools

import jax
import jax.numpy as jnp
from jax import lax
from jax.experimental import pallas as pl
from jax.experimental.pallas import tpu as pltpu
from jax.experimental.pallas import tpu_sc as plsc

N = 4096
D = 2048
E = 8
EP = 128
TG = 512
TM = 256
SP = N // TM + E - 1
NP = SP * TM


def _gating_body(x_ref, wg_ref, gate_ref, sel_ref, rank_ref,
                 psum_ref, csum_ref):
    i = pl.program_id(0)
    x = x_ref[...]
    wg = wg_ref[...]
    logits = jnp.dot(x, wg, preferred_element_type=jnp.float32)
    col = lax.broadcasted_iota(jnp.int32, (TG, EP), 1)
    lm = jnp.where(col < E, logits, jnp.float32(-1e30))
    mx = jnp.max(lm, axis=1, keepdims=True)
    ex = jnp.where(col < E, jnp.exp(lm - mx), 0.0)
    den = jnp.sum(ex, axis=1, keepdims=True)
    probs = ex / den
    pmax = jnp.max(probs, axis=1, keepdims=True)
    cand = jnp.where(probs >= pmax, col, EP)
    gate = jnp.min(cand, axis=1)
    gate_ref[0, 0, :] = gate
    sel_ref[0, 0, :] = pmax[:, 0]
    onehot = (col == gate[:, None]).astype(jnp.float32)

    @pl.when(i == 0)
    def _():
        psum_ref[...] = jnp.zeros_like(psum_ref)
        csum_ref[...] = jnp.zeros_like(csum_ref)

    r0 = lax.broadcasted_iota(jnp.int32, (TG, TG), 0)
    c0 = lax.broadcasted_iota(jnp.int32, (TG, TG), 1)
    tri = (c0 <= r0).astype(jnp.float32)
    cums = jnp.dot(tri, onehot, preferred_element_type=jnp.float32)
    base = csum_ref[0, :]
    rankf = jnp.sum(onehot * (base[None, :] + cums - 1.0), axis=1)
    rank_ref[0, 0, :] = rankf.astype(jnp.int32)

    psum_ref[...] += jnp.sum(probs, axis=0, keepdims=True)
    csum_ref[...] += jnp.sum(onehot, axis=0, keepdims=True)


def _gating(xf, wg_pad):
    g = N // TG
    return pl.pallas_call(
        _gating_body,
        grid=(g,),
        in_specs=[
            pl.BlockSpec((TG, D), lambda i: (i, 0)),
            pl.BlockSpec((D, EP), lambda i: (0, 0)),
        ],
        out_specs=[
            pl.BlockSpec((1, 1, TG), lambda i: (i, 0, 0)),
            pl.BlockSpec((1, 1, TG), lambda i: (i, 0, 0)),
            pl.BlockSpec((1, 1, TG), lambda i: (i, 0, 0)),
            pl.BlockSpec((1, EP), lambda i: (0, 0)),
            pl.BlockSpec((1, EP), lambda i: (0, 0)),
        ],
        out_shape=[
            jax.ShapeDtypeStruct((g, 1, TG), jnp.int32),
            jax.ShapeDtypeStruct((g, 1, TG), jnp.float32),
            jax.ShapeDtypeStruct((g, 1, TG), jnp.int32),
            jax.ShapeDtypeStruct((1, EP), jnp.float32),
            jax.ShapeDtypeStruct((1, EP), jnp.float32),
        ],
    )(xf, wg_pad)


def _route_metadata(gate, rank, counts):
    ntiles = (counts + TM - 1) // TM
    tile_cum = jnp.cumsum(ntiles)
    pstart = TM * (tile_cum - ntiles)
    pos = jnp.take(pstart, gate) + rank
    e_ids = jnp.searchsorted(
        tile_cum, jnp.arange(SP, dtype=jnp.int32), side='right')
    e_ids = jnp.minimum(e_ids, E - 1).astype(jnp.int32)
    return pos, e_ids.reshape(1, SP)


def _sc_scatter_rows(src, pos):
    info = plsc.get_sparse_core_info()
    nw = info.num_cores * info.num_subcores
    per_w = N // nw
    ch = 16
    mesh = plsc.VectorSubcoreMesh(core_axis_name="c", subcore_axis_name="s")

    @functools.partial(
        pl.kernel, mesh=mesh,
        out_type=jax.ShapeDtypeStruct((NP, D), jnp.float32),
        scratch_types=[
            pltpu.VMEM((ch,), jnp.int32),
            pltpu.VMEM((ch,), jnp.int32),
            pltpu.VMEM((ch, D), jnp.float32),
            pltpu.VMEM((ch, D), jnp.float32),
            pltpu.SemaphoreType.DMA,
            pltpu.SemaphoreType.DMA,
        ],
    )
    def k(src_hbm, pos_hbm, out_hbm, idx_a, idx_b, rows_a, rows_b,
          sem_a, sem_b):
        wid = lax.axis_index("s") * info.num_cores + lax.axis_index("c")
        base = wid * per_w
        idx_v = (idx_a, idx_b)
        rows_v = (rows_a, rows_b)
        sems = (sem_a, sem_b)
        pend = [None, None]
        for c in range(per_w // ch):
            b = c & 1
            if pend[b] is not None:
                pend[b].wait()
            off = base + c * ch
            pltpu.sync_copy(pos_hbm.at[pl.ds(off, ch)], idx_v[b])
            pltpu.sync_copy(src_hbm.at[pl.ds(off, ch)], rows_v[b])
            pend[b] = pltpu.async_copy(rows_v[b], out_hbm.at[idx_v[b]],
                                       sems[b])
        pend[0].wait()
        pend[1].wait()

    return k(src, pos)


def _sc_gather(table, idx, n_out):
    info = plsc.get_sparse_core_info()
    nw = info.num_cores * info.num_subcores
    per_w = n_out // nw
    ch = 16
    mesh = plsc.VectorSubcoreMesh(core_axis_name="c", subcore_axis_name="s")

    @functools.partial(
        pl.kernel, mesh=mesh,
        out_type=jax.ShapeDtypeStruct((n_out, D), jnp.float32),
        scratch_types=[
            pltpu.VMEM((ch,), jnp.int32),
            pltpu.VMEM((ch,), jnp.int32),
            pltpu.VMEM((ch, D), jnp.float32),
            pltpu.VMEM((ch, D), jnp.float32),
            pltpu.SemaphoreType.DMA,
            pltpu.SemaphoreType.DMA,
            pltpu.SemaphoreType.DMA,
            pltpu.SemaphoreType.DMA,
        ],
    )
    def k(table_hbm, idx_hbm, out_hbm, idx_a, idx_b, rows_a, rows_b,
          sem_ra, sem_rb, sem_wa, sem_wb):
        wid = lax.axis_index("s") * info.num_cores + lax.axis_index("c")
        base = wid * per_w
        idx_v = (idx_a, idx_b)
        rows_v = (rows_a, rows_b)
        sem_r = (sem_ra, sem_rb)
        sem_w = (sem_wa, sem_wb)
        pend = [None, None]
        for c in range(per_w // ch):
            b = c & 1
            if pend[b] is not None:
                pend[b].wait()
            off = base + c * ch
            pltpu.sync_copy(idx_hbm.at[pl.ds(off, ch)], idx_v[b])
            pltpu.async_copy(table_hbm.at[idx_v[b]], rows_v[b],
                             sem_r[b]).wait()
            pend[b] = pltpu.async_copy(rows_v[b], out_hbm.at[pl.ds(off, ch)],
                                       sem_w[b])
        pend[0].wait()
        pend[1].wait()

    return k(table, idx)


OB = D // 2


def _tile_body(eids_ref, xs_ref, we0_ref, we1_ref, be_ref, sel_ref, out_ref):
    scale = sel_ref[0, 0, :]
    xm = (xs_ref[...] * scale[:, None]).astype(jnp.bfloat16)
    bias = scale[:, None] * be_ref[0, 0, :][None, :]
    for half, w_ref in ((0, we0_ref), (1, we1_ref)):
        w = w_ref[0].astype(jnp.bfloat16)
        contrib = lax.dot_general(
            xm, w, (((1,), (1,)), ((), ())),
            preferred_element_type=jnp.float32)
        out_ref[:, half * OB:(half + 1) * OB] = (
            contrib + bias[:, half * OB:(half + 1) * OB])


def _tile_matmul(xs, we, be, sel3, e_ids):
    grid_spec = pltpu.PrefetchScalarGridSpec(
        num_scalar_prefetch=1,
        grid=(SP,),
        in_specs=[
            pl.BlockSpec((TM, D), lambda s, m: (s, 0)),
            pl.BlockSpec((1, OB, D), lambda s, m: (m[0, s], 0, 0)),
            pl.BlockSpec((1, OB, D), lambda s, m: (m[0, s], 1, 0)),
            pl.BlockSpec((1, 1, D), lambda s, m: (m[0, s], 0, 0)),
            pl.BlockSpec((1, 1, TM), lambda s, m: (s, 0, 0)),
        ],
        out_specs=pl.BlockSpec((TM, D), lambda s, m: (s, 0)),
    )
    return pl.pallas_call(
        _tile_body,
        grid_spec=grid_spec,
        out_shape=jax.ShapeDtypeStruct((NP, D), jnp.float32),
    )(e_ids, xs, we, we, be, sel3)


def kernel(x, attention_mask, Wg, We, be):
    del attention_mask
    bsz, seq_len, dim = x.shape
    xf = x.reshape(N, D)
    wg_pad = jnp.zeros((D, EP), jnp.float32).at[:, :E].set(Wg.T)

    gate3, sel3g, rank3, psum, csum = _gating(xf, wg_pad)
    gate = gate3.reshape(N)
    sel = sel3g.reshape(N)
    rank = rank3.reshape(N)
    counts = csum[0, :E].astype(jnp.int32)

    pos, e_ids = _route_metadata(gate, rank, counts)

    xs = _sc_scatter_rows(xf, pos)
    sel_s3 = jnp.zeros((NP,), jnp.float32).at[pos].set(sel).reshape(SP, 1, TM)
    ys = _tile_matmul(xs, We, be.reshape(E, 1, D), sel_s3, e_ids)
    out_tok = _sc_gather(ys, pos, N)

    out = out_tok.reshape(bsz, seq_len, dim)
    probs_mean = psum[0, :E] / jnp.float32(N)
    f = counts.astype(jnp.float32) / jnp.float32(N)
    balance_loss = jnp.float32(E) * jnp.sum(probs_mean * f)
    return (out, balance_loss, counts)


# speedup vs baseline: 1.0325x; 1.0325x over previous
import functools

import jax
import jax.numpy as jnp
from jax import lax
from jax.experimental import pallas as pl
from jax.experimental.pallas import tpu as pltpu
from jax.experimental.pallas import tpu_sc as plsc

N = 4096
D = 2048
E = 8
EP = 128
TG = 512
TM = 256
SP = N // TM + E - 1
NP = SP * TM


def _gating_body(x_ref, wg_ref, gate_ref, sel_ref, rank_ref,
                 psum_ref, csum_ref):
    i = pl.program_id(0)
    x = x_ref[...]
    wg = wg_ref[...]
    logits = jnp.dot(x, wg, preferred_element_type=jnp.float32)
    col = lax.broadcasted_iota(jnp.int32, (TG, EP), 1)
    lm = jnp.where(col < E, logits, jnp.float32(-1e30))
    mx = jnp.max(lm, axis=1, keepdims=True)
    ex = jnp.where(col < E, jnp.exp(lm - mx), 0.0)
    den = jnp.sum(ex, axis=1, keepdims=True)
    probs = ex / den
    pmax = jnp.max(probs, axis=1, keepdims=True)
    cand = jnp.where(probs >= pmax, col, EP)
    gate = jnp.min(cand, axis=1)
    gate_ref[0, 0, :] = gate
    sel_ref[0, 0, :] = pmax[:, 0]
    onehot = (col == gate[:, None]).astype(jnp.float32)

    @pl.when(i == 0)
    def _():
        psum_ref[...] = jnp.zeros_like(psum_ref)
        csum_ref[...] = jnp.zeros_like(csum_ref)

    r0 = lax.broadcasted_iota(jnp.int32, (TG, TG), 0)
    c0 = lax.broadcasted_iota(jnp.int32, (TG, TG), 1)
    tri = (c0 <= r0).astype(jnp.float32)
    cums = jnp.dot(tri, onehot, preferred_element_type=jnp.float32)
    base = csum_ref[0, :]
    rankf = jnp.sum(onehot * (base[None, :] + cums - 1.0), axis=1)
    rank_ref[0, 0, :] = rankf.astype(jnp.int32)

    psum_ref[...] += jnp.sum(probs, axis=0, keepdims=True)
    csum_ref[...] += jnp.sum(onehot, axis=0, keepdims=True)


def _gating(xf, wg_pad):
    g = N // TG
    return pl.pallas_call(
        _gating_body,
        grid=(g,),
        in_specs=[
            pl.BlockSpec((TG, D), lambda i: (i, 0)),
            pl.BlockSpec((D, EP), lambda i: (0, 0)),
        ],
        out_specs=[
            pl.BlockSpec((1, 1, TG), lambda i: (i, 0, 0)),
            pl.BlockSpec((1, 1, TG), lambda i: (i, 0, 0)),
            pl.BlockSpec((1, 1, TG), lambda i: (i, 0, 0)),
            pl.BlockSpec((1, EP), lambda i: (0, 0)),
            pl.BlockSpec((1, EP), lambda i: (0, 0)),
        ],
        out_shape=[
            jax.ShapeDtypeStruct((g, 1, TG), jnp.int32),
            jax.ShapeDtypeStruct((g, 1, TG), jnp.float32),
            jax.ShapeDtypeStruct((g, 1, TG), jnp.int32),
            jax.ShapeDtypeStruct((1, EP), jnp.float32),
            jax.ShapeDtypeStruct((1, EP), jnp.float32),
        ],
    )(xf, wg_pad)


def _route_metadata(gate, rank, counts):
    ntiles = (counts + TM - 1) // TM
    tile_cum = jnp.cumsum(ntiles)
    pstart = TM * (tile_cum - ntiles)
    pos = jnp.take(pstart, gate) + rank
    e_ids = jnp.searchsorted(
        tile_cum, jnp.arange(SP, dtype=jnp.int32), side='right')
    e_ids = jnp.minimum(e_ids, E - 1).astype(jnp.int32)
    return pos, e_ids.reshape(1, SP)


def _sc_scatter_rows(src, pos):
    info = plsc.get_sparse_core_info()
    nw = info.num_cores * info.num_subcores
    per_w = N // nw
    ch = 32
    mesh = plsc.VectorSubcoreMesh(core_axis_name="c", subcore_axis_name="s")

    @functools.partial(
        pl.kernel, mesh=mesh,
        out_type=jax.ShapeDtypeStruct((NP, D), jnp.float32),
        scratch_types=[
            pltpu.VMEM((ch,), jnp.int32),
            pltpu.VMEM((ch, D), jnp.float32),
            pltpu.SemaphoreType.DMA,
        ],
    )
    def k(src_hbm, pos_hbm, out_hbm, idx_v, rows_v, sem):
        wid = lax.axis_index("s") * info.num_cores + lax.axis_index("c")
        base = wid * per_w
        for c in range(per_w // ch):
            off = base + c * ch
            pltpu.sync_copy(pos_hbm.at[pl.ds(off, ch)], idx_v)
            pltpu.sync_copy(src_hbm.at[pl.ds(off, ch)], rows_v)
            pltpu.async_copy(rows_v, out_hbm.at[idx_v], sem).wait()

    return k(src, pos)


def _sc_gather(table, idx, n_out):
    info = plsc.get_sparse_core_info()
    nw = info.num_cores * info.num_subcores
    per_w = n_out // nw
    ch = 32
    mesh = plsc.VectorSubcoreMesh(core_axis_name="c", subcore_axis_name="s")

    @functools.partial(
        pl.kernel, mesh=mesh,
        out_type=jax.ShapeDtypeStruct((n_out, D), jnp.float32),
        scratch_types=[
            pltpu.VMEM((ch,), jnp.int32),
            pltpu.VMEM((ch, D), jnp.float32),
            pltpu.SemaphoreType.DMA,
        ],
    )
    def k(table_hbm, idx_hbm, out_hbm, idx_v, rows_v, sem):
        wid = lax.axis_index("s") * info.num_cores + lax.axis_index("c")
        base = wid * per_w
        for c in range(per_w // ch):
            off = base + c * ch
            pltpu.sync_copy(idx_hbm.at[pl.ds(off, ch)], idx_v)
            pltpu.async_copy(table_hbm.at[idx_v], rows_v, sem).wait()
            pltpu.sync_copy(rows_v, out_hbm.at[pl.ds(off, ch)])

    return k(table, idx)


OB = D // 2


def _tile_body(eids_ref, xs_ref, we0_ref, we1_ref, be_ref, sel_ref, out_ref):
    scale = sel_ref[0, 0, :]
    xm = (xs_ref[...] * scale[:, None]).astype(jnp.bfloat16)
    bias = scale[:, None] * be_ref[0, 0, :][None, :]
    for half, w_ref in ((0, we0_ref), (1, we1_ref)):
        w = w_ref[0].astype(jnp.bfloat16)
        contrib = lax.dot_general(
            xm, w, (((1,), (1,)), ((), ())),
            preferred_element_type=jnp.float32)
        out_ref[:, half * OB:(half + 1) * OB] = (
            contrib + bias[:, half * OB:(half + 1) * OB])


def _tile_matmul(xs, we, be, sel3, e_ids):
    grid_spec = pltpu.PrefetchScalarGridSpec(
        num_scalar_prefetch=1,
        grid=(SP,),
        in_specs=[
            pl.BlockSpec((TM, D), lambda s, m: (s, 0)),
            pl.BlockSpec((1, OB, D), lambda s, m: (m[0, s], 0, 0)),
            pl.BlockSpec((1, OB, D), lambda s, m: (m[0, s], 1, 0)),
            pl.BlockSpec((1, 1, D), lambda s, m: (m[0, s], 0, 0)),
            pl.BlockSpec((1, 1, TM), lambda s, m: (s, 0, 0)),
        ],
        out_specs=pl.BlockSpec((TM, D), lambda s, m: (s, 0)),
    )
    return pl.pallas_call(
        _tile_body,
        grid_spec=grid_spec,
        out_shape=jax.ShapeDtypeStruct((NP, D), jnp.float32),
    )(e_ids, xs, we, we, be, sel3)


def kernel(x, attention_mask, Wg, We, be):
    del attention_mask
    bsz, seq_len, dim = x.shape
    xf = x.reshape(N, D)
    wg_pad = jnp.zeros((D, EP), jnp.float32).at[:, :E].set(Wg.T)

    gate3, sel3g, rank3, psum, csum = _gating(xf, wg_pad)
    gate = gate3.reshape(N)
    sel = sel3g.reshape(N)
    rank = rank3.reshape(N)
    counts = csum[0, :E].astype(jnp.int32)

    pos, e_ids = _route_metadata(gate, rank, counts)

    xs = _sc_scatter_rows(xf, pos)
    sel_s3 = jnp.zeros((NP,), jnp.float32).at[pos].set(sel).reshape(SP, 1, TM)
    ys = _tile_matmul(xs, We, be.reshape(E, 1, D), sel_s3, e_ids)
    out_tok = _sc_gather(ys, pos, N)

    out = out_tok.reshape(bsz, seq_len, dim)
    probs_mean = psum[0, :E] / jnp.float32(N)
    f = counts.astype(jnp.float32) / jnp.float32(N)
    balance_loss = jnp.float32(E) * jnp.sum(probs_mean * f)
    return (out, balance_loss, counts)
